# Initial kernel scaffold; baseline (speedup 1.0000x reference)
import jax
import jax.numpy as jnp
from jax.experimental import pallas as pl


def kernel(x, tables, W1, b1, W2, b2):
    raise NotImplementedError("write your pallas kernel here")


# speedup vs baseline: 27.7234x; 27.7234x over previous
import functools

import jax
import jax.numpy as jnp
from jax.experimental import pallas as pl
from jax.experimental.pallas import tpu as pltpu

_B, _P, _E, _V, _NT = 1024, 20, 64, 100, 33
_NF = _NT + 2
_VP = 128
_R = 640
_H = 256


def _proj_kernel(t_ref, w_ref, o_ref):
    o_ref[0] = jax.lax.dot(
        t_ref[0], w_ref[0],
        precision=jax.lax.Precision.HIGHEST,
        preferred_element_type=jnp.float32,
    )


def _ffn_kernel(idx_ref, xd_ref, tb_ref, u_ref, b1_ref, w2_ref, o_ref, oh_ref):
    lane = jax.lax.broadcasted_iota(jnp.int32, (_R, _VP), 1)
    for f in range(_NT):
        oh = (lane == idx_ref[:, f : f + 1]).astype(jnp.bfloat16)
        oh_ref[:, f * _VP : (f + 1) * _VP] = oh
    acc = jax.lax.dot(
        oh_ref[...], tb_ref[...],
        precision=jax.lax.Precision.DEFAULT,
        preferred_element_type=jnp.float32,
    )
    acc = acc + xd_ref[:, 0:1] * u_ref[0:1, :] + xd_ref[:, 1:2] * u_ref[1:2, :]
    h = jnp.maximum(acc + b1_ref[0:1, :], 0.0)
    logit = jnp.sum(h * w2_ref[0:1, :], axis=1, keepdims=True)

    m = jnp.max(logit)
    e = jnp.exp(logit - m)
    ngrp = _R // _P
    r_i = jax.lax.broadcasted_iota(jnp.int32, (ngrp, _R), 1)
    g_i = jax.lax.broadcasted_iota(jnp.int32, (ngrp, _R), 0)
    gsum = (r_i // _P == g_i).astype(jnp.float32)
    r_j = jax.lax.broadcasted_iota(jnp.int32, (_R, ngrp), 0)
    g_j = jax.lax.broadcasted_iota(jnp.int32, (_R, ngrp), 1)
    gbk = (r_j // _P == g_j).astype(jnp.float32)
    sums = jax.lax.dot(gsum, e, precision=jax.lax.Precision.HIGHEST,
                       preferred_element_type=jnp.float32)
    denom = jax.lax.dot(gbk, sums, precision=jax.lax.Precision.HIGHEST,
                        preferred_element_type=jnp.float32)
    o_ref[...] = e / denom


def kernel(x, tables, W1, b1, W2, b2):
    n = _B * _P
    idx = x[:, :, 2:].astype(jnp.int32).reshape(n, _NT)
    xd = x[:, :, :2].reshape(n, 2)

    t_aug = jnp.zeros((_NF, _VP, _E), jnp.float32)
    t_aug = t_aug.at[0:2].set(1.0)
    t_aug = t_aug.at[2:, :_V, :].set(tables)
    w1e = W1.reshape(_H, _NF, _E).transpose(1, 2, 0)

    tproj = pl.pallas_call(
        _proj_kernel,
        grid=(_NF,),
        in_specs=[
            pl.BlockSpec((1, _VP, _E), lambda f: (f, 0, 0)),
            pl.BlockSpec((1, _E, _H), lambda f: (f, 0, 0)),
        ],
        out_specs=pl.BlockSpec((1, _VP, _H), lambda f: (f, 0, 0)),
        out_shape=jax.ShapeDtypeStruct((_NF, _VP, _H), jnp.float32),
    )(t_aug, w1e)

    u01 = tproj[0:2, 0, :]
    tb = tproj[2:].reshape(_NT * _VP, _H).astype(jnp.bfloat16)

    out = pl.pallas_call(
        _ffn_kernel,
        grid=(n // _R,),
        in_specs=[
            pl.BlockSpec((_R, _NT), lambda i: (i, 0)),
            pl.BlockSpec((_R, 2), lambda i: (i, 0)),
            pl.BlockSpec((_NT * _VP, _H), lambda i: (0, 0)),
            pl.BlockSpec((2, _H), lambda i: (0, 0)),
            pl.BlockSpec((1, _H), lambda i: (0, 0)),
            pl.BlockSpec((1, _H), lambda i: (0, 0)),
        ],
        out_specs=pl.BlockSpec((_R, 1), lambda i: (i, 0)),
        out_shape=jax.ShapeDtypeStruct((n, 1), jnp.float32),
        scratch_shapes=[pltpu.VMEM((_R, _NT * _VP), jnp.bfloat16)],
    )(idx, xd, tb, u01, b1.reshape(1, _H), W2, )

    return out.reshape(_B, _P, 1)


# speedup vs baseline: 43.9189x; 1.5842x over previous
import functools

import jax
import jax.numpy as jnp
from jax.experimental import pallas as pl
from jax.experimental.pallas import tpu as pltpu

_B, _P, _E, _V, _NT = 1024, 20, 64, 100, 33
_NF = _NT + 2
_VP = 112
_R = 640
_H = 256
_K = _NT * _VP


def _proj_kernel(w_ref, t_ref, o_ref):
    o_ref[0] = jax.lax.dot(
        w_ref[0], t_ref[0],
        precision=jax.lax.Precision.HIGHEST,
        preferred_element_type=jnp.float32,
    )


def _ffn_kernel(idx_ref, xd_ref, tb_ref, u_ref, b1_ref, w2_ref, o_ref, oh_ref):
    sub = jax.lax.broadcasted_iota(jnp.int32, (_VP, _R), 0)
    for f in range(_NT):
        oh = (sub == idx_ref[f : f + 1, :]).astype(jnp.bfloat16)
        oh_ref[f * _VP : (f + 1) * _VP, :] = oh
    acc = jax.lax.dot(
        tb_ref[...], oh_ref[...],
        precision=jax.lax.Precision.DEFAULT,
        preferred_element_type=jnp.float32,
    )
    acc = acc + u_ref[:, 0:1] * xd_ref[0:1, :] + u_ref[:, 1:2] * xd_ref[1:2, :]
    h = jnp.maximum(acc + b1_ref[...], 0.0)
    o_ref[0] = jnp.sum(h * w2_ref[...], axis=0, keepdims=True)


def _softmax_kernel(l_ref, o_ref):
    l = l_ref[...]
    m = jnp.max(l, axis=1, keepdims=True)
    e = jnp.exp(l - m)
    o_ref[...] = e / jnp.sum(e, axis=1, keepdims=True)


def kernel(x, tables, W1, b1, W2, b2):
    n = _B * _P
    idx_t = x[:, :, 2:].astype(jnp.int32).reshape(n, _NT).T
    xd_t = x[:, :, :2].reshape(n, 2).T

    tt = jnp.zeros((_NF, _E, _VP), jnp.float32)
    tt = tt.at[0:2].set(1.0)
    tt = tt.at[2:, :, :_V].set(tables.transpose(0, 2, 1))
    w1t = W1.reshape(_H, _NF, _E).transpose(1, 0, 2)

    tproj = pl.pallas_call(
        _proj_kernel,
        grid=(_NF,),
        in_specs=[
            pl.BlockSpec((1, _H, _E), lambda f: (f, 0, 0)),
            pl.BlockSpec((1, _E, _VP), lambda f: (f, 0, 0)),
        ],
        out_specs=pl.BlockSpec((1, _H, _VP), lambda f: (f, 0, 0)),
        out_shape=jax.ShapeDtypeStruct((_NF, _H, _VP), jnp.float32),
    )(w1t, tt)

    u01 = tproj[0:2, :, 0].T
    tb = tproj[2:].transpose(1, 0, 2).reshape(_H, _K).astype(jnp.bfloat16)

    nblk = n // _R
    logits = pl.pallas_call(
        _ffn_kernel,
        grid=(nblk,),
        in_specs=[
            pl.BlockSpec((_NT, _R), lambda i: (0, i)),
            pl.BlockSpec((2, _R), lambda i: (0, i)),
            pl.BlockSpec((_H, _K), lambda i: (0, 0)),
            pl.BlockSpec((_H, 2), lambda i: (0, 0)),
            pl.BlockSpec((_H, 1), lambda i: (0, 0)),
            pl.BlockSpec((_H, 1), lambda i: (0, 0)),
        ],
        out_specs=pl.BlockSpec((1, 1, _R), lambda i: (i, 0, 0)),
        out_shape=jax.ShapeDtypeStruct((nblk, 1, _R), jnp.float32),
        scratch_shapes=[pltpu.VMEM((_K, _R), jnp.bfloat16)],
    )(idx_t, xd_t, tb, u01, b1.reshape(_H, 1), W2.reshape(_H, 1))

    out = pl.pallas_call(
        _softmax_kernel,
        out_shape=jax.ShapeDtypeStruct((_B, _P), jnp.float32),
    )(logits.reshape(_B, _P))

    return out.reshape(_B, _P, 1)


# speedup vs baseline: 45.5199x; 1.0365x over previous
import functools

import jax
import jax.numpy as jnp
from jax.experimental import pallas as pl
from jax.experimental.pallas import tpu as pltpu

_B, _P, _E, _V, _NT = 1024, 20, 64, 100, 33
_NF = _NT + 2
_VP = 112
_R = 640
_H = 256
_K = _NT * _VP


def _proj_kernel(w_ref, t_ref, o_ref):
    o_ref[0] = jax.lax.dot(
        w_ref[0], t_ref[0],
        precision=jax.lax.Precision.DEFAULT,
        preferred_element_type=jnp.float32,
    )


def _ffn_kernel(idx_ref, xd_ref, tb_ref, u_ref, b1_ref, w2_ref, o_ref, oh_ref):
    sub = jax.lax.broadcasted_iota(jnp.int32, (_VP, _R), 0)
    for f in range(_NT):
        oh = (sub == idx_ref[f : f + 1, :]).astype(jnp.bfloat16)
        oh_ref[f * _VP : (f + 1) * _VP, :] = oh
    acc = jax.lax.dot(
        tb_ref[...], oh_ref[...],
        precision=jax.lax.Precision.DEFAULT,
        preferred_element_type=jnp.float32,
    )
    acc = acc + u_ref[:, 0:1] * xd_ref[0:1, :] + u_ref[:, 1:2] * xd_ref[1:2, :]
    h = jnp.maximum(acc + b1_ref[...], 0.0)
    o_ref[0] = jax.lax.dot(
        w2_ref[...], h.astype(jnp.bfloat16),
        precision=jax.lax.Precision.DEFAULT,
        preferred_element_type=jnp.float32,
    )


def _softmax_kernel(l_ref, o_ref):
    l = l_ref[...]
    m = jnp.max(l, axis=1, keepdims=True)
    e = jnp.exp(l - m)
    o_ref[...] = e / jnp.sum(e, axis=1, keepdims=True)


def _logits(x, tables, W1, b1, W2, b2):
    n = _B * _P
    idx_t = x[:, :, 2:].astype(jnp.int32).reshape(n, _NT).T
    xd_t = x[:, :, :2].reshape(n, 2).T

    tt = jnp.zeros((_NF, _E, _VP), jnp.float32)
    tt = tt.at[0:2].set(1.0)
    tt = tt.at[2:, :, :_V].set(tables.transpose(0, 2, 1))
    w1t = W1.reshape(_H, _NF, _E).transpose(1, 0, 2)

    proj = pl.pallas_call(
        _proj_kernel,
        grid=(_NF,),
        in_specs=[
            pl.BlockSpec((1, _H, _E), lambda f: (f, 0, 0)),
            pl.BlockSpec((1, _E, _VP), lambda f: (f, 0, 0)),
        ],
        out_specs=pl.BlockSpec((1, _H, _VP), lambda f: (f, 0, 0)),
        out_shape=jax.ShapeDtypeStruct((_NF, _H, _VP), jnp.float32),
    )(w1t.astype(jnp.bfloat16), tt.astype(jnp.bfloat16))

    u01 = proj[0:2, :, 0].T
    tb = proj[2:].transpose(1, 0, 2).reshape(_H, _K).astype(jnp.bfloat16)
    w2b = W2.astype(jnp.bfloat16)

    nblk = n // _R
    logits = pl.pallas_call(
        _ffn_kernel,
        grid=(nblk,),
        in_specs=[
            pl.BlockSpec((_NT, _R), lambda i: (0, i)),
            pl.BlockSpec((2, _R), lambda i: (0, i)),
            pl.BlockSpec((_H, _K), lambda i: (0, 0)),
            pl.BlockSpec((_H, 2), lambda i: (0, 0)),
            pl.BlockSpec((_H, 1), lambda i: (0, 0)),
            pl.BlockSpec((1, _H), lambda i: (0, 0)),
        ],
        out_specs=pl.BlockSpec((1, 1, _R), lambda i: (i, 0, 0)),
        out_shape=jax.ShapeDtypeStruct((nblk, 1, _R), jnp.float32),
        scratch_shapes=[pltpu.VMEM((_K, _R), jnp.bfloat16)],
    )(idx_t, xd_t, tb, u01, b1.reshape(_H, 1), w2b)
    return logits


def kernel(x, tables, W1, b1, W2, b2):
    logits = _logits(x, tables, W1, b1, W2, b2)
    out = pl.pallas_call(
        _softmax_kernel,
        out_shape=jax.ShapeDtypeStruct((_B, _P), jnp.float32),
    )(logits.reshape(_B, _P))

    return out.reshape(_B, _P, 1)
